# baseline (speedup 1.0000x reference)
import functools

import jax
import jax.numpy as jnp
import numpy as np
from jax import lax
from jax.experimental import pallas as pl
from jax.experimental.pallas import tpu as pltpu
from jax.experimental.pallas import tpu_sc as plsc

_B, _L, _K, _H = 2, 2048, 32, 128
_SCALE = 30.0
_N = _B * _L
_E = _N * _K
_T = 256
_G = _N // _T
_F32 = jnp.float32
_BF16 = jnp.bfloat16

_NC, _NS = 2, 16
_NW = _NC * _NS
_CHUNK = 128
_ROWS_PER_W = _E // _NW
_NCH = _ROWS_PER_W // _CHUNK


def _gelu(x):
    return 0.5 * x * (1.0 + lax.erf(x * 0.7071067811865476))


def _ln(x, g, o):
    m = x.mean(-1, keepdims=True)
    v = jnp.mean(jnp.square(x - m), -1, keepdims=True)
    return (x - m) * lax.rsqrt(v + 1e-5) * g + o


def _mm(a, w):
    return jnp.dot(a.astype(_BF16), w.astype(_BF16), preferred_element_type=_F32)


_HI_MASK = np.uint32(0xFFFF0000)


def _pack_bf16(x):
    a = x[:, : _H // 2].astype(_BF16).astype(_F32)
    b = x[:, _H // 2:].astype(_BF16).astype(_F32)
    ai = lax.bitcast_convert_type(a, jnp.uint32)
    bi = lax.bitcast_convert_type(b, jnp.uint32)
    return lax.bitcast_convert_type(
        lax.shift_right_logical(ai, np.uint32(16)) | (bi & _HI_MASK), jnp.int32)


def _unpack_bf16(w):
    wu = lax.bitcast_convert_type(w, jnp.uint32)
    lo = lax.bitcast_convert_type(lax.shift_left(wu, np.uint32(16)), _F32)
    hi = lax.bitcast_convert_type(wu & _HI_MASK, _F32)
    return jnp.concatenate([lo, hi], axis=-1)


def _proj_body(hv_ref, w1_ref, b1_ref, q_ref, p_ref):
    hv = hv_ref[...].reshape(_N, _H)
    w1 = w1_ref[...]
    q_ref[...] = _mm(hv, w1[:_H]) + b1_ref[...]
    p_ref[...] = _mm(hv, w1[_H:2 * _H])


def _project(hv3d, w1, b1):
    return pl.pallas_call(
        _proj_body,
        out_shape=(
            jax.ShapeDtypeStruct((_N, _H), _F32),
            jax.ShapeDtypeStruct((_N, _H), _F32),
        ),
    )(hv3d, w1, b1)


def _gather_body(table_hbm, idx_hbm, out_hbm, idx_v, bufa, bufb, gsa, gsb, ssa, ssb):
    cid = lax.axis_index("c")
    sid = lax.axis_index("s")
    wid = sid * _NC + cid
    base = wid * _ROWS_PER_W
    pltpu.sync_copy(idx_hbm.at[wid], idx_v)
    pltpu.async_copy(table_hbm.at[idx_v.at[0]], bufa, gsa)

    def pair(i, carry):
        c0 = i * 2
        c1 = c0 + 1
        pltpu.make_async_copy(table_hbm.at[idx_v.at[c0]], bufa, gsa).wait()
        sa = pltpu.async_copy(bufa, out_hbm.at[pl.ds(base + c0 * _CHUNK, _CHUNK)], ssa)
        pltpu.async_copy(table_hbm.at[idx_v.at[c1]], bufb, gsb)
        sa.wait()

        @pl.when(i < _NCH // 2 - 1)
        def _():
            pltpu.async_copy(table_hbm.at[idx_v.at[c0 + 2]], bufa, gsa)

        pltpu.make_async_copy(table_hbm.at[idx_v.at[c1]], bufb, gsb).wait()
        pltpu.async_copy(bufb, out_hbm.at[pl.ds(base + c1 * _CHUNK, _CHUNK)], ssb).wait()
        return carry

    lax.fori_loop(0, _NCH // 2, pair, 0)


@functools.cache
def _gather_call():
    return pl.kernel(
        _gather_body,
        out_type=jax.ShapeDtypeStruct((_E, _H), _F32),
        mesh=plsc.VectorSubcoreMesh(
            core_axis_name="c", subcore_axis_name="s",
            num_cores=_NC, num_subcores=_NS),
        scratch_types=[
            pltpu.VMEM((_NCH, _CHUNK), jnp.int32),
            pltpu.VMEM((_CHUNK, _H), _F32),
            pltpu.VMEM((_CHUNK, _H), _F32),
            pltpu.SemaphoreType.DMA,
            pltpu.SemaphoreType.DMA,
            pltpu.SemaphoreType.DMA,
            pltpu.SemaphoreType.DMA,
        ],
    )


def _sc_gather(table, idx3d):
    return _gather_call()(table, idx3d)


def _node_body(he_ref, gp1_ref, q1_ref, hv_ref, ma_ref, mv_ref,
               w1_ref, w2_ref, b2_ref, w3_ref, b3_ref,
               g1_ref, o1_ref, wdi_ref, bdi_ref, wdo_ref, bdo_ref,
               g2_ref, o2_ref, w11_ref, b11_ref,
               hv_out_ref, q2_ref, p2_ref):
    tk = _T * _K
    he2 = he_ref[...].reshape(tk, _H)
    x = _mm(he2, w1_ref[...][2 * _H:])
    x = x + gp1_ref[...]
    q1 = q1_ref[...]
    x = x + jnp.broadcast_to(q1[:, None, :], (_T, _K, _H)).reshape(tk, _H)
    x = _gelu(x)
    x = _gelu(_mm(x, w2_ref[...]) + b2_ref[...])
    x = _mm(x, w3_ref[...]) + b3_ref[...]
    x = x * ma_ref[...]
    dh = x.reshape(_T, _K, _H).sum(axis=1) * (1.0 / _SCALE)
    h1 = _ln(hv_ref[...].reshape(_T, _H) + dh, g1_ref[...], o1_ref[...])
    f = _gelu(_mm(h1, wdi_ref[...]) + bdi_ref[...])
    f = _mm(f, wdo_ref[...]) + bdo_ref[...]
    h2 = _ln(h1 + f, g2_ref[...], o2_ref[...])
    h2 = h2 * mv_ref[...]
    hv_out_ref[...] = h2.reshape(1, _T, _H)
    w11 = w11_ref[...]
    q2_ref[...] = _mm(h2, w11[:_H]) + b11_ref[...]
    p2_ref[...] = _mm(h2, w11[_H:2 * _H])


_LT = _L // _T


def _node_update(he4d, gp1, q1, hv3d, ma_flat, mv_col, pp):
    flat = lambda b, i: (b * _LT + i, 0)
    const2 = lambda b, i: (0, 0)
    node_spec = pl.BlockSpec((_T, _H), flat)
    param_specs = [pl.BlockSpec(p.shape, const2) for p in pp]
    return pl.pallas_call(
        _node_body,
        grid=(_B, _LT),
        in_specs=[
            pl.BlockSpec((1, _T, _K, _H), lambda b, i: (b, i, 0, 0)),
            pl.BlockSpec((_T * _K, _H), flat),
            node_spec,
            pl.BlockSpec((1, _T, _H), lambda b, i: (b, i, 0)),
            pl.BlockSpec((_T * _K, 1), flat),
            pl.BlockSpec((_T, 1), flat),
            *param_specs,
        ],
        out_specs=(pl.BlockSpec((1, _T, _H), lambda b, i: (b, i, 0)),
                   node_spec, node_spec),
        out_shape=(
            jax.ShapeDtypeStruct((_B, _L, _H), _F32),
            jax.ShapeDtypeStruct((_N, _H), _F32),
            jax.ShapeDtypeStruct((_N, _H), _F32),
        ),
    )(he4d, gp1, q1, hv3d, ma_flat, mv_col, *pp)


def _edge_body(he_ref, gp2_ref, q2_ref,
               w11_ref, w12_ref, b12_ref, w13_ref, b13_ref, g3_ref, o3_ref,
               he_out_ref):
    tk = _T * _K
    he2 = he_ref[...].reshape(tk, _H)
    x = _mm(he2, w11_ref[...][2 * _H:])
    x = x + gp2_ref[...]
    q2 = q2_ref[...]
    x = x + jnp.broadcast_to(q2[:, None, :], (_T, _K, _H)).reshape(tk, _H)
    x = _gelu(x)
    x = _gelu(_mm(x, w12_ref[...]) + b12_ref[...])
    x = _mm(x, w13_ref[...]) + b13_ref[...]
    y = _ln(he2 + x, g3_ref[...], o3_ref[...])
    he_out_ref[...] = y.reshape(1, _T, _K, _H)


def _edge_update(he4d, gp2, q2, pp):
    flat = lambda b, i: (b * _LT + i, 0)
    const2 = lambda b, i: (0, 0)
    edge_spec = pl.BlockSpec((1, _T, _K, _H), lambda b, i: (b, i, 0, 0))
    param_specs = [pl.BlockSpec(p.shape, const2) for p in pp]
    return pl.pallas_call(
        _edge_body,
        grid=(_B, _LT),
        in_specs=[edge_spec, pl.BlockSpec((_T * _K, _H), flat),
                  pl.BlockSpec((_T, _H), flat), *param_specs],
        out_specs=edge_spec,
        out_shape=jax.ShapeDtypeStruct((_B, _L, _K, _H), _F32),
    )(he4d, gp2, q2, *pp)


def kernel(h_V, h_E, E_idx, mask_V, mask_attend, params):
    p = params
    w1 = p["W1"]["w"]
    w11 = p["W11"]["w"]
    row1 = lambda b: b.reshape(1, -1)

    ma_flat = mask_attend.reshape(_E, 1)
    mv_col = mask_V.reshape(_N, 1)
    flat_idx = (E_idx + (jnp.arange(_B, dtype=jnp.int32) * _L)[:, None, None])
    idx3d = flat_idx.reshape(_NW, _NCH, _CHUNK)

    q1, p1 = _project(h_V, w1, row1(p["W1"]["b"]))
    gp1 = _sc_gather(p1, idx3d)

    node_params = [
        w1, p["W2"]["w"], row1(p["W2"]["b"]), p["W3"]["w"], row1(p["W3"]["b"]),
        row1(p["ln1"]["g"]), row1(p["ln1"]["o"]),
        p["Wd_in"]["w"], row1(p["Wd_in"]["b"]), p["Wd_out"]["w"], row1(p["Wd_out"]["b"]),
        row1(p["ln2"]["g"]), row1(p["ln2"]["o"]),
        w11, row1(p["W11"]["b"]),
    ]
    hv_new, q2, p2 = _node_update(h_E, gp1, q1, h_V, ma_flat, mv_col, node_params)

    gp2 = _sc_gather(p2, idx3d)

    edge_params = [
        w11, p["W12"]["w"], row1(p["W12"]["b"]), p["W13"]["w"], row1(p["W13"]["b"]),
        row1(p["ln3"]["g"]), row1(p["ln3"]["o"]),
    ]
    he_new = _edge_update(h_E, gp2, q2, edge_params)

    return (hv_new, he_new)


# speedup vs baseline: 1.1187x; 1.1187x over previous
import functools

import jax
import jax.numpy as jnp
import numpy as np
from jax import lax
from jax.experimental import pallas as pl
from jax.experimental.pallas import tpu as pltpu
from jax.experimental.pallas import tpu_sc as plsc

_B, _L, _K, _H = 2, 2048, 32, 128
_SCALE = 30.0
_N = _B * _L
_E = _N * _K
_T = 256
_G = _N // _T
_F32 = jnp.float32
_BF16 = jnp.bfloat16

_NC, _NS = 2, 16
_NW = _NC * _NS
_CHUNK = 128
_ROWS_PER_W = _E // _NW
_NCH = _ROWS_PER_W // _CHUNK


def _gelu(x):
    return 0.5 * x * (1.0 + lax.erf(x * 0.7071067811865476))


def _ln(x, g, o):
    m = x.mean(-1, keepdims=True)
    v = jnp.mean(jnp.square(x - m), -1, keepdims=True)
    return (x - m) * lax.rsqrt(v + 1e-5) * g + o


def _mm(a, w):
    return jnp.dot(a.astype(_BF16), w.astype(_BF16), preferred_element_type=_F32)


_HI_MASK = np.uint32(0xFFFF0000)


def _pack_bf16(x):
    a = x[:, : _H // 2].astype(_BF16).astype(_F32)
    b = x[:, _H // 2:].astype(_BF16).astype(_F32)
    ai = lax.bitcast_convert_type(a, jnp.uint32)
    bi = lax.bitcast_convert_type(b, jnp.uint32)
    return lax.bitcast_convert_type(
        lax.shift_right_logical(ai, np.uint32(16)) | (bi & _HI_MASK), jnp.int32)


def _unpack_bf16(w):
    wu = lax.bitcast_convert_type(w, jnp.uint32)
    lo = lax.bitcast_convert_type(lax.shift_left(wu, np.uint32(16)), _F32)
    hi = lax.bitcast_convert_type(wu & _HI_MASK, _F32)
    return jnp.concatenate([lo, hi], axis=-1)


def _proj_body(hv_ref, w1_ref, b1_ref, q_ref, p_ref):
    hv = hv_ref[...].reshape(_N, _H)
    w1 = w1_ref[...]
    q_ref[...] = _mm(hv, w1[:_H]) + b1_ref[...]
    p_ref[...] = _mm(hv, w1[_H:2 * _H])


def _project(hv3d, w1, b1):
    return pl.pallas_call(
        _proj_body,
        out_shape=(
            jax.ShapeDtypeStruct((_N, _H), _F32),
            jax.ShapeDtypeStruct((_N, _H), _F32),
        ),
    )(hv3d, w1, b1)


def _gather_body(table_hbm, idx_hbm, out_hbm, idx_v, bufa, bufb, gsa, gsb, ssa, ssb):
    cid = lax.axis_index("c")
    sid = lax.axis_index("s")
    wid = sid * _NC + cid
    base = wid * _ROWS_PER_W
    pltpu.sync_copy(idx_hbm.at[wid], idx_v)
    pltpu.async_copy(table_hbm.at[idx_v.at[0]], bufa, gsa)

    def pair(i, carry):
        c0 = i * 2
        c1 = c0 + 1
        pltpu.make_async_copy(table_hbm.at[idx_v.at[c0]], bufa, gsa).wait()
        sa = pltpu.async_copy(bufa, out_hbm.at[pl.ds(base + c0 * _CHUNK, _CHUNK)], ssa)
        pltpu.async_copy(table_hbm.at[idx_v.at[c1]], bufb, gsb)
        sa.wait()

        @pl.when(i < _NCH // 2 - 1)
        def _():
            pltpu.async_copy(table_hbm.at[idx_v.at[c0 + 2]], bufa, gsa)

        pltpu.make_async_copy(table_hbm.at[idx_v.at[c1]], bufb, gsb).wait()
        pltpu.async_copy(bufb, out_hbm.at[pl.ds(base + c1 * _CHUNK, _CHUNK)], ssb).wait()
        return carry

    lax.fori_loop(0, _NCH // 2, pair, 0)


@functools.cache
def _gather_call():
    return pl.kernel(
        _gather_body,
        out_type=jax.ShapeDtypeStruct((_E, _H), _F32),
        mesh=plsc.VectorSubcoreMesh(
            core_axis_name="c", subcore_axis_name="s",
            num_cores=_NC, num_subcores=_NS),
        scratch_types=[
            pltpu.VMEM((_NCH, _CHUNK), jnp.int32),
            pltpu.VMEM((_CHUNK, _H), _F32),
            pltpu.VMEM((_CHUNK, _H), _F32),
            pltpu.SemaphoreType.DMA,
            pltpu.SemaphoreType.DMA,
            pltpu.SemaphoreType.DMA,
            pltpu.SemaphoreType.DMA,
        ],
    )


def _sc_gather(table, idx3d):
    return _gather_call()(table, idx3d)


def _node_body(he_ref, gp1_ref, q1_ref, hv_ref, ma_ref, mv_ref,
               w1_ref, w2_ref, b2_ref, w3_ref, b3_ref,
               g1_ref, o1_ref, wdi_ref, bdi_ref, wdo_ref, bdo_ref,
               g2_ref, o2_ref, w11_ref, b11_ref,
               hv_out_ref, q2_ref, p2_ref):
    tk = _T * _K
    he2 = he_ref[...].reshape(tk, _H)
    x = _mm(he2, w1_ref[...][2 * _H:])
    x = x + gp1_ref[...]
    q1 = q1_ref[...]
    x = x + jnp.broadcast_to(q1[:, None, :], (_T, _K, _H)).reshape(tk, _H)
    x = _gelu(x)
    x = _gelu(_mm(x, w2_ref[...]) + b2_ref[...])
    x = _mm(x, w3_ref[...]) + b3_ref[...]
    x3 = x.reshape(_T, _K, _H) * ma_ref[...].reshape(_T, _K)[:, :, None]
    dh = x3.sum(axis=1) * (1.0 / _SCALE)
    h1 = _ln(hv_ref[...].reshape(_T, _H) + dh, g1_ref[...], o1_ref[...])
    f = _gelu(_mm(h1, wdi_ref[...]) + bdi_ref[...])
    f = _mm(f, wdo_ref[...]) + bdo_ref[...]
    h2 = _ln(h1 + f, g2_ref[...], o2_ref[...])
    h2 = h2 * mv_ref[...]
    hv_out_ref[...] = h2.reshape(1, _T, _H)
    w11 = w11_ref[...]
    q2_ref[...] = _mm(h2, w11[:_H]) + b11_ref[...]
    p2_ref[...] = _mm(h2, w11[_H:2 * _H])


_LT = _L // _T


def _node_update(he4d, gp1, q1, hv3d, ma3d, mv_col, pp):
    flat = lambda b, i: (b * _LT + i, 0)
    const2 = lambda b, i: (0, 0)
    node_spec = pl.BlockSpec((_T, _H), flat)
    param_specs = [pl.BlockSpec(p.shape, const2) for p in pp]
    return pl.pallas_call(
        _node_body,
        grid=(_B, _LT),
        in_specs=[
            pl.BlockSpec((1, _T, _K, _H), lambda b, i: (b, i, 0, 0)),
            pl.BlockSpec((_T * _K, _H), flat),
            node_spec,
            pl.BlockSpec((1, _T, _H), lambda b, i: (b, i, 0)),
            pl.BlockSpec((1, _T, _K), lambda b, i: (b, i, 0)),
            pl.BlockSpec((_T, 1), flat),
            *param_specs,
        ],
        out_specs=(pl.BlockSpec((1, _T, _H), lambda b, i: (b, i, 0)),
                   node_spec, node_spec),
        out_shape=(
            jax.ShapeDtypeStruct((_B, _L, _H), _F32),
            jax.ShapeDtypeStruct((_N, _H), _F32),
            jax.ShapeDtypeStruct((_N, _H), _F32),
        ),
    )(he4d, gp1, q1, hv3d, ma3d, mv_col, *pp)


def _edge_body(he_ref, gp2_ref, q2_ref,
               w11_ref, w12_ref, b12_ref, w13_ref, b13_ref, g3_ref, o3_ref,
               he_out_ref):
    tk = _T * _K
    he2 = he_ref[...].reshape(tk, _H)
    x = _mm(he2, w11_ref[...][2 * _H:])
    x = x + gp2_ref[...]
    q2 = q2_ref[...]
    x = x + jnp.broadcast_to(q2[:, None, :], (_T, _K, _H)).reshape(tk, _H)
    x = _gelu(x)
    x = _gelu(_mm(x, w12_ref[...]) + b12_ref[...])
    x = _mm(x, w13_ref[...]) + b13_ref[...]
    y = _ln(he2 + x, g3_ref[...], o3_ref[...])
    he_out_ref[...] = y.reshape(1, _T, _K, _H)


def _edge_update(he4d, gp2, q2, pp):
    flat = lambda b, i: (b * _LT + i, 0)
    const2 = lambda b, i: (0, 0)
    edge_spec = pl.BlockSpec((1, _T, _K, _H), lambda b, i: (b, i, 0, 0))
    param_specs = [pl.BlockSpec(p.shape, const2) for p in pp]
    return pl.pallas_call(
        _edge_body,
        grid=(_B, _LT),
        in_specs=[edge_spec, pl.BlockSpec((_T * _K, _H), flat),
                  pl.BlockSpec((_T, _H), flat), *param_specs],
        out_specs=edge_spec,
        out_shape=jax.ShapeDtypeStruct((_B, _L, _K, _H), _F32),
    )(he4d, gp2, q2, *pp)


def kernel(h_V, h_E, E_idx, mask_V, mask_attend, params):
    p = params
    w1 = p["W1"]["w"]
    w11 = p["W11"]["w"]
    row1 = lambda b: b.reshape(1, -1)

    mv_col = mask_V.reshape(_N, 1)
    flat_idx = (E_idx + (jnp.arange(_B, dtype=jnp.int32) * _L)[:, None, None])
    idx3d = flat_idx.reshape(_NW, _NCH, _CHUNK)

    q1, p1 = _project(h_V, w1, row1(p["W1"]["b"]))
    gp1 = _sc_gather(p1, idx3d)

    node_params = [
        w1, p["W2"]["w"], row1(p["W2"]["b"]), p["W3"]["w"], row1(p["W3"]["b"]),
        row1(p["ln1"]["g"]), row1(p["ln1"]["o"]),
        p["Wd_in"]["w"], row1(p["Wd_in"]["b"]), p["Wd_out"]["w"], row1(p["Wd_out"]["b"]),
        row1(p["ln2"]["g"]), row1(p["ln2"]["o"]),
        w11, row1(p["W11"]["b"]),
    ]
    hv_new, q2, p2 = _node_update(h_E, gp1, q1, h_V, mask_attend, mv_col, node_params)

    gp2 = _sc_gather(p2, idx3d)

    edge_params = [
        w11, p["W12"]["w"], row1(p["W12"]["b"]), p["W13"]["w"], row1(p["W13"]["b"]),
        row1(p["ln3"]["g"]), row1(p["ln3"]["o"]),
    ]
    he_new = _edge_update(h_E, gp2, q2, edge_params)

    return (hv_new, he_new)


# speedup vs baseline: 1.1345x; 1.0141x over previous
import functools

import jax
import jax.numpy as jnp
import numpy as np
from jax import lax
from jax.experimental import pallas as pl
from jax.experimental.pallas import tpu as pltpu
from jax.experimental.pallas import tpu_sc as plsc

_B, _L, _K, _H = 2, 2048, 32, 128
_SCALE = 30.0
_N = _B * _L
_E = _N * _K
_T = 256
_G = _N // _T
_F32 = jnp.float32
_BF16 = jnp.bfloat16

_NC, _NS = 2, 16
_NW = _NC * _NS
_CHUNK = 128
_ROWS_PER_W = _E // _NW
_NCH = _ROWS_PER_W // _CHUNK


def _gelu(x):
    return 0.5 * x * (1.0 + lax.erf(x * 0.7071067811865476))


def _ln(x, g, o):
    m = x.mean(-1, keepdims=True)
    v = jnp.mean(jnp.square(x - m), -1, keepdims=True)
    return (x - m) * lax.rsqrt(v + 1e-5) * g + o


def _mm(a, w):
    return jnp.dot(a.astype(_BF16), w.astype(_BF16), preferred_element_type=_F32)


_HI_MASK = np.uint32(0xFFFF0000)


def _pack_bf16(x):
    a = x[:, : _H // 2].astype(_BF16).astype(_F32)
    b = x[:, _H // 2:].astype(_BF16).astype(_F32)
    ai = lax.bitcast_convert_type(a, jnp.uint32)
    bi = lax.bitcast_convert_type(b, jnp.uint32)
    return lax.bitcast_convert_type(
        lax.shift_right_logical(ai, np.uint32(16)) | (bi & _HI_MASK), jnp.int32)


def _unpack_bf16(w):
    wu = lax.bitcast_convert_type(w, jnp.uint32)
    lo = lax.bitcast_convert_type(lax.shift_left(wu, np.uint32(16)), _F32)
    hi = lax.bitcast_convert_type(wu & _HI_MASK, _F32)
    return jnp.concatenate([lo, hi], axis=-1)


def _proj_body(hv_ref, w1_ref, b1_ref, q_ref, p_ref):
    hv = hv_ref[...].reshape(_N, _H)
    w1 = w1_ref[...]
    q_ref[...] = _mm(hv, w1[:_H]) + b1_ref[...]
    p_ref[...] = _mm(hv, w1[_H:2 * _H])


def _project(hv3d, w1, b1):
    return pl.pallas_call(
        _proj_body,
        out_shape=(
            jax.ShapeDtypeStruct((_N, _H), _F32),
            jax.ShapeDtypeStruct((_N, _H), _F32),
        ),
    )(hv3d, w1, b1)


def _gather_body(table_hbm, idx_hbm, out_hbm, idx_v,
                 b0, b1, b2, b3, g0, g1, g2, g3, s0, s1, s2, s3):
    cid = lax.axis_index("c")
    sid = lax.axis_index("s")
    wid = sid * _NC + cid
    base = wid * _ROWS_PER_W
    bufs = (b0, b1, b2, b3)
    gs = (g0, g1, g2, g3)
    ss = (s0, s1, s2, s3)
    pltpu.sync_copy(idx_hbm.at[wid], idx_v)
    for j in range(3):
        pltpu.async_copy(table_hbm.at[idx_v.at[j]], bufs[j], gs[j])

    def quad(i, carry):
        for j in range(4):
            c = i * 4 + j
            jm = (j + 3) % 4
            pltpu.make_async_copy(table_hbm.at[idx_v.at[c]], bufs[j], gs[j]).wait()
            pltpu.async_copy(bufs[j], out_hbm.at[pl.ds(base + c * _CHUNK, _CHUNK)], ss[j])

            @pl.when(c >= 1)
            def _():
                pltpu.make_async_copy(
                    bufs[jm], out_hbm.at[pl.ds(base + (c - 1) * _CHUNK, _CHUNK)],
                    ss[jm]).wait()

            @pl.when(c + 3 < _NCH)
            def _():
                pltpu.async_copy(table_hbm.at[idx_v.at[c + 3]], bufs[jm], gs[jm])

        return carry

    lax.fori_loop(0, _NCH // 4, quad, 0)
    pltpu.make_async_copy(
        bufs[3], out_hbm.at[pl.ds(base + (_NCH - 1) * _CHUNK, _CHUNK)], ss[3]).wait()


@functools.cache
def _gather_call():
    return pl.kernel(
        _gather_body,
        out_type=jax.ShapeDtypeStruct((_E, _H), _F32),
        mesh=plsc.VectorSubcoreMesh(
            core_axis_name="c", subcore_axis_name="s",
            num_cores=_NC, num_subcores=_NS),
        scratch_types=[
            pltpu.VMEM((_NCH, _CHUNK), jnp.int32),
            pltpu.VMEM((_CHUNK, _H), _F32),
            pltpu.VMEM((_CHUNK, _H), _F32),
            pltpu.VMEM((_CHUNK, _H), _F32),
            pltpu.VMEM((_CHUNK, _H), _F32),
            pltpu.SemaphoreType.DMA,
            pltpu.SemaphoreType.DMA,
            pltpu.SemaphoreType.DMA,
            pltpu.SemaphoreType.DMA,
            pltpu.SemaphoreType.DMA,
            pltpu.SemaphoreType.DMA,
            pltpu.SemaphoreType.DMA,
            pltpu.SemaphoreType.DMA,
        ],
    )


def _sc_gather(table, idx3d):
    return _gather_call()(table, idx3d)


def _node_body(he_ref, gp1_ref, q1_ref, hv_ref, ma_ref, mv_ref,
               w1_ref, w2_ref, b2_ref, w3_ref, b3_ref,
               g1_ref, o1_ref, wdi_ref, bdi_ref, wdo_ref, bdo_ref,
               g2_ref, o2_ref, w11_ref, b11_ref,
               hv_out_ref, q2_ref, p2_ref):
    tk = _T * _K
    he2 = he_ref[...].reshape(tk, _H)
    x = _mm(he2, w1_ref[...][2 * _H:])
    x = x + gp1_ref[...]
    q1 = q1_ref[...]
    x = x + jnp.broadcast_to(q1[:, None, :], (_T, _K, _H)).reshape(tk, _H)
    x = _gelu(x)
    x = _gelu(_mm(x, w2_ref[...]) + b2_ref[...])
    x = _mm(x, w3_ref[...]) + b3_ref[...]
    x3 = x.reshape(_T, _K, _H) * ma_ref[...].reshape(_T, _K)[:, :, None]
    dh = x3.sum(axis=1) * (1.0 / _SCALE)
    h1 = _ln(hv_ref[...].reshape(_T, _H) + dh, g1_ref[...], o1_ref[...])
    f = _gelu(_mm(h1, wdi_ref[...]) + bdi_ref[...])
    f = _mm(f, wdo_ref[...]) + bdo_ref[...]
    h2 = _ln(h1 + f, g2_ref[...], o2_ref[...])
    h2 = h2 * mv_ref[...]
    hv_out_ref[...] = h2.reshape(1, _T, _H)
    w11 = w11_ref[...]
    q2_ref[...] = _mm(h2, w11[:_H]) + b11_ref[...]
    p2_ref[...] = _mm(h2, w11[_H:2 * _H])


_LT = _L // _T


def _node_update(he4d, gp1, q1, hv3d, ma3d, mv_col, pp):
    flat = lambda b, i: (b * _LT + i, 0)
    const2 = lambda b, i: (0, 0)
    node_spec = pl.BlockSpec((_T, _H), flat)
    param_specs = [pl.BlockSpec(p.shape, const2) for p in pp]
    return pl.pallas_call(
        _node_body,
        grid=(_B, _LT),
        in_specs=[
            pl.BlockSpec((1, _T, _K, _H), lambda b, i: (b, i, 0, 0)),
            pl.BlockSpec((_T * _K, _H), flat),
            node_spec,
            pl.BlockSpec((1, _T, _H), lambda b, i: (b, i, 0)),
            pl.BlockSpec((1, _T, _K), lambda b, i: (b, i, 0)),
            pl.BlockSpec((_T, 1), flat),
            *param_specs,
        ],
        out_specs=(pl.BlockSpec((1, _T, _H), lambda b, i: (b, i, 0)),
                   node_spec, node_spec),
        out_shape=(
            jax.ShapeDtypeStruct((_B, _L, _H), _F32),
            jax.ShapeDtypeStruct((_N, _H), _F32),
            jax.ShapeDtypeStruct((_N, _H), _F32),
        ),
    )(he4d, gp1, q1, hv3d, ma3d, mv_col, *pp)


def _edge_body(he_ref, gp2_ref, q2_ref,
               w11_ref, w12_ref, b12_ref, w13_ref, b13_ref, g3_ref, o3_ref,
               he_out_ref):
    tk = _T * _K
    he2 = he_ref[...].reshape(tk, _H)
    x = _mm(he2, w11_ref[...][2 * _H:])
    x = x + gp2_ref[...]
    q2 = q2_ref[...]
    x = x + jnp.broadcast_to(q2[:, None, :], (_T, _K, _H)).reshape(tk, _H)
    x = _gelu(x)
    x = _gelu(_mm(x, w12_ref[...]) + b12_ref[...])
    x = _mm(x, w13_ref[...]) + b13_ref[...]
    y = _ln(he2 + x, g3_ref[...], o3_ref[...])
    he_out_ref[...] = y.reshape(1, _T, _K, _H)


def _edge_update(he4d, gp2, q2, pp):
    flat = lambda b, i: (b * _LT + i, 0)
    const2 = lambda b, i: (0, 0)
    edge_spec = pl.BlockSpec((1, _T, _K, _H), lambda b, i: (b, i, 0, 0))
    param_specs = [pl.BlockSpec(p.shape, const2) for p in pp]
    return pl.pallas_call(
        _edge_body,
        grid=(_B, _LT),
        in_specs=[edge_spec, pl.BlockSpec((_T * _K, _H), flat),
                  pl.BlockSpec((_T, _H), flat), *param_specs],
        out_specs=edge_spec,
        out_shape=jax.ShapeDtypeStruct((_B, _L, _K, _H), _F32),
    )(he4d, gp2, q2, *pp)


def kernel(h_V, h_E, E_idx, mask_V, mask_attend, params):
    p = params
    w1 = p["W1"]["w"]
    w11 = p["W11"]["w"]
    row1 = lambda b: b.reshape(1, -1)

    mv_col = mask_V.reshape(_N, 1)
    flat_idx = (E_idx + (jnp.arange(_B, dtype=jnp.int32) * _L)[:, None, None])
    idx3d = flat_idx.reshape(_NW, _NCH, _CHUNK)

    q1, p1 = _project(h_V, w1, row1(p["W1"]["b"]))
    gp1 = _sc_gather(p1, idx3d)

    node_params = [
        w1, p["W2"]["w"], row1(p["W2"]["b"]), p["W3"]["w"], row1(p["W3"]["b"]),
        row1(p["ln1"]["g"]), row1(p["ln1"]["o"]),
        p["Wd_in"]["w"], row1(p["Wd_in"]["b"]), p["Wd_out"]["w"], row1(p["Wd_out"]["b"]),
        row1(p["ln2"]["g"]), row1(p["ln2"]["o"]),
        w11, row1(p["W11"]["b"]),
    ]
    hv_new, q2, p2 = _node_update(h_E, gp1, q1, h_V, mask_attend, mv_col, node_params)

    gp2 = _sc_gather(p2, idx3d)

    edge_params = [
        w11, p["W12"]["w"], row1(p["W12"]["b"]), p["W13"]["w"], row1(p["W13"]["b"]),
        row1(p["ln3"]["g"]), row1(p["ln3"]["o"]),
    ]
    he_new = _edge_update(h_E, gp2, q2, edge_params)

    return (hv_new, he_new)


# speedup vs baseline: 1.1385x; 1.0035x over previous
import functools

import jax
import jax.numpy as jnp
from jax import lax
from jax.experimental import pallas as pl
from jax.experimental.pallas import tpu as pltpu
from jax.experimental.pallas import tpu_sc as plsc

_B, _L, _K, _H = 2, 2048, 32, 128
_SCALE = 30.0
_N = _B * _L
_E = _N * _K
_T = 256
_G = _N // _T
_F32 = jnp.float32
_BF16 = jnp.bfloat16

_NC, _NS = 2, 16
_NW = _NC * _NS
_CHUNK = 128
_ROWS_PER_W = _E // _NW
_NCH = _ROWS_PER_W // _CHUNK


def _gelu(x):
    return 0.5 * x * (1.0 + lax.erf(x * 0.7071067811865476))


def _ln(x, g, o):
    m = x.mean(-1, keepdims=True)
    v = jnp.mean(jnp.square(x - m), -1, keepdims=True)
    return (x - m) * lax.rsqrt(v + 1e-5) * g + o


def _mm(a, w):
    return jnp.dot(a.astype(_BF16), w.astype(_BF16), preferred_element_type=_F32)


def _proj_body(hv_ref, w1_ref, b1_ref, q_ref, p_ref):
    hv = hv_ref[...].reshape(_N, _H)
    w1 = w1_ref[...]
    q_ref[...] = _mm(hv, w1[:_H]) + b1_ref[...]
    p_ref[...] = _mm(hv, w1[_H:2 * _H])


def _project(hv3d, w1, b1):
    return pl.pallas_call(
        _proj_body,
        out_shape=(
            jax.ShapeDtypeStruct((_N, _H), _F32),
            jax.ShapeDtypeStruct((_N, _H), _F32),
        ),
    )(hv3d, w1, b1)


def _gather_body(table_hbm, idx_hbm, out_hbm, idx_v,
                 b0, b1, b2, b3, g0, g1, g2, g3, s0, s1, s2, s3):
    cid = lax.axis_index("c")
    sid = lax.axis_index("s")
    wid = sid * _NC + cid
    base = wid * _ROWS_PER_W
    bufs = (b0, b1, b2, b3)
    gs = (g0, g1, g2, g3)
    ss = (s0, s1, s2, s3)
    pltpu.sync_copy(idx_hbm.at[wid], idx_v)
    for j in range(3):
        pltpu.async_copy(table_hbm.at[idx_v.at[j]], bufs[j], gs[j])

    def quad(i, carry):
        for j in range(4):
            c = i * 4 + j
            jm = (j + 3) % 4
            pltpu.make_async_copy(table_hbm.at[idx_v.at[c]], bufs[j], gs[j]).wait()
            pltpu.async_copy(bufs[j], out_hbm.at[pl.ds(base + c * _CHUNK, _CHUNK)], ss[j])

            @pl.when(c >= 1)
            def _():
                pltpu.make_async_copy(
                    bufs[jm], out_hbm.at[pl.ds(base + (c - 1) * _CHUNK, _CHUNK)],
                    ss[jm]).wait()

            @pl.when(c + 3 < _NCH)
            def _():
                pltpu.async_copy(table_hbm.at[idx_v.at[c + 3]], bufs[jm], gs[jm])

        return carry

    lax.fori_loop(0, _NCH // 4, quad, 0)
    pltpu.make_async_copy(
        bufs[3], out_hbm.at[pl.ds(base + (_NCH - 1) * _CHUNK, _CHUNK)], ss[3]).wait()


@functools.cache
def _gather_call():
    return pl.kernel(
        _gather_body,
        out_type=jax.ShapeDtypeStruct((_E, _H), _F32),
        mesh=plsc.VectorSubcoreMesh(
            core_axis_name="c", subcore_axis_name="s",
            num_cores=_NC, num_subcores=_NS),
        scratch_types=[
            pltpu.VMEM((_NCH, _CHUNK), jnp.int32),
            pltpu.VMEM((_CHUNK, _H), _F32),
            pltpu.VMEM((_CHUNK, _H), _F32),
            pltpu.VMEM((_CHUNK, _H), _F32),
            pltpu.VMEM((_CHUNK, _H), _F32),
            pltpu.SemaphoreType.DMA,
            pltpu.SemaphoreType.DMA,
            pltpu.SemaphoreType.DMA,
            pltpu.SemaphoreType.DMA,
            pltpu.SemaphoreType.DMA,
            pltpu.SemaphoreType.DMA,
            pltpu.SemaphoreType.DMA,
            pltpu.SemaphoreType.DMA,
        ],
    )


def _sc_gather(table, idx3d):
    return _gather_call()(table, idx3d)


def _node_body(he_ref, gp1_ref, q1_ref, hv_ref, ma_ref, mv_ref,
               w1_ref, w2_ref, b2_ref, w3_ref, b3_ref,
               g1_ref, o1_ref, wdi_ref, bdi_ref, wdo_ref, bdo_ref,
               g2_ref, o2_ref, w11_ref, b11_ref,
               hv_out_ref, q2_ref, p2_ref):
    tk = _T * _K
    he2 = he_ref[...].reshape(tk, _H)
    x = _mm(he2, w1_ref[...][2 * _H:])
    x = x + gp1_ref[...]
    q1 = q1_ref[...]
    x = x + jnp.broadcast_to(q1[:, None, :], (_T, _K, _H)).reshape(tk, _H)
    x = _gelu(x)
    x = _gelu(_mm(x, w2_ref[...]) + b2_ref[...])
    x = _mm(x, w3_ref[...]) + b3_ref[...]
    x3 = x.reshape(_T, _K, _H) * ma_ref[...].reshape(_T, _K)[:, :, None]
    dh = x3.sum(axis=1) * (1.0 / _SCALE)
    h1 = _ln(hv_ref[...].reshape(_T, _H) + dh, g1_ref[...], o1_ref[...])
    f = _gelu(_mm(h1, wdi_ref[...]) + bdi_ref[...])
    f = _mm(f, wdo_ref[...]) + bdo_ref[...]
    h2 = _ln(h1 + f, g2_ref[...], o2_ref[...])
    h2 = h2 * mv_ref[...]
    hv_out_ref[...] = h2.reshape(1, _T, _H)
    w11 = w11_ref[...]
    q2_ref[...] = _mm(h2, w11[:_H]) + b11_ref[...]
    p2_ref[...] = _mm(h2, w11[_H:2 * _H])


_LT = _L // _T


def _node_update(he4d, gp1, q1, hv3d, ma3d, mv_col, pp):
    flat = lambda b, i: (b * _LT + i, 0)
    const2 = lambda b, i: (0, 0)
    node_spec = pl.BlockSpec((_T, _H), flat)
    param_specs = [pl.BlockSpec(p.shape, const2) for p in pp]
    return pl.pallas_call(
        _node_body,
        grid=(_B, _LT),
        in_specs=[
            pl.BlockSpec((1, _T, _K, _H), lambda b, i: (b, i, 0, 0)),
            pl.BlockSpec((_T * _K, _H), flat),
            node_spec,
            pl.BlockSpec((1, _T, _H), lambda b, i: (b, i, 0)),
            pl.BlockSpec((1, _T, _K), lambda b, i: (b, i, 0)),
            pl.BlockSpec((_T, 1), flat),
            *param_specs,
        ],
        out_specs=(pl.BlockSpec((1, _T, _H), lambda b, i: (b, i, 0)),
                   node_spec, node_spec),
        out_shape=(
            jax.ShapeDtypeStruct((_B, _L, _H), _F32),
            jax.ShapeDtypeStruct((_N, _H), _F32),
            jax.ShapeDtypeStruct((_N, _H), _F32),
        ),
    )(he4d, gp1, q1, hv3d, ma3d, mv_col, *pp)


def _edge_body(he_ref, gp2_ref, q2_ref,
               w11_ref, w12_ref, b12_ref, w13_ref, b13_ref, g3_ref, o3_ref,
               he_out_ref):
    tk = _T * _K
    he2 = he_ref[...].reshape(tk, _H)
    x = _mm(he2, w11_ref[...][2 * _H:])
    x = x + gp2_ref[...]
    q2 = q2_ref[...]
    x = x + jnp.broadcast_to(q2[:, None, :], (_T, _K, _H)).reshape(tk, _H)
    x = _gelu(x)
    x = _gelu(_mm(x, w12_ref[...]) + b12_ref[...])
    x = _mm(x, w13_ref[...]) + b13_ref[...]
    y = _ln(he2 + x, g3_ref[...], o3_ref[...])
    he_out_ref[...] = y.reshape(1, _T, _K, _H)


def _edge_update(he4d, gp2, q2, pp):
    flat = lambda b, i: (b * _LT + i, 0)
    const2 = lambda b, i: (0, 0)
    edge_spec = pl.BlockSpec((1, _T, _K, _H), lambda b, i: (b, i, 0, 0))
    param_specs = [pl.BlockSpec(p.shape, const2) for p in pp]
    return pl.pallas_call(
        _edge_body,
        grid=(_B, _LT),
        in_specs=[edge_spec, pl.BlockSpec((_T * _K, _H), flat),
                  pl.BlockSpec((_T, _H), flat), *param_specs],
        out_specs=edge_spec,
        out_shape=jax.ShapeDtypeStruct((_B, _L, _K, _H), _F32),
    )(he4d, gp2, q2, *pp)


def kernel(h_V, h_E, E_idx, mask_V, mask_attend, params):
    p = params
    w1 = p["W1"]["w"]
    w11 = p["W11"]["w"]
    row1 = lambda b: b.reshape(1, -1)

    mv_col = mask_V.reshape(_N, 1)
    flat_idx = (E_idx + (jnp.arange(_B, dtype=jnp.int32) * _L)[:, None, None])
    idx3d = flat_idx.reshape(_NW, _NCH, _CHUNK)

    q1, p1 = _project(h_V, w1, row1(p["W1"]["b"]))
    gp1 = _sc_gather(p1, idx3d)

    node_params = [
        w1, p["W2"]["w"], row1(p["W2"]["b"]), p["W3"]["w"], row1(p["W3"]["b"]),
        row1(p["ln1"]["g"]), row1(p["ln1"]["o"]),
        p["Wd_in"]["w"], row1(p["Wd_in"]["b"]), p["Wd_out"]["w"], row1(p["Wd_out"]["b"]),
        row1(p["ln2"]["g"]), row1(p["ln2"]["o"]),
        w11, row1(p["W11"]["b"]),
    ]
    hv_new, q2, p2 = _node_update(h_E, gp1, q1, h_V, mask_attend, mv_col, node_params)

    gp2 = _sc_gather(p2, idx3d)

    edge_params = [
        w11, p["W12"]["w"], row1(p["W12"]["b"]), p["W13"]["w"], row1(p["W13"]["b"]),
        row1(p["ln3"]["g"]), row1(p["ln3"]["o"]),
    ]
    he_new = _edge_update(h_E, gp2, q2, edge_params)

    return (hv_new, he_new)
